# baseline (speedup 1.0000x reference)
import functools

import jax
import jax.numpy as jnp
from jax import lax
from jax.experimental import pallas as pl
from jax.experimental.pallas import tpu as pltpu
from jax.experimental.pallas import tpu_sc as plsc

NC = 2
NS = 16
NW = NC * NS

B, L = 4096, 50
D = 128
TOTAL = B * L
PER_W = TOTAL // NW
CHUNK = 64
NCHUNK = PER_W // CHUNK
NBUF = 7
GAHEAD = 5

_mesh = plsc.VectorSubcoreMesh(
    core_axis_name="c", subcore_axis_name="s", num_cores=NC, num_subcores=NS
)


@functools.partial(
    pl.kernel,
    out_type=jax.ShapeDtypeStruct((B, L * D), jnp.float32),
    mesh=_mesh,
    compiler_params=pltpu.CompilerParams(use_tc_tiling_on_sc=True),
    scratch_types=[
        pltpu.VMEM((L, 2, CHUNK), jnp.int32),
        pltpu.VMEM((NBUF, CHUNK, D), jnp.float32),
        tuple(pltpu.SemaphoreType.DMA for _ in range(NBUF)),
        tuple(pltpu.SemaphoreType.DMA for _ in range(NBUF)),
    ],
)
def _gather_kernel(ids_hbm, table_hbm, out_hbm, idx_v, bufs, gsem, wsem):
    wid = lax.axis_index("s") * NC + lax.axis_index("c")
    row_base = wid * (B // NW)
    nhead = (GAHEAD + 1) // 2
    pltpu.sync_copy(ids_hbm.at[pl.ds(0, nhead), wid], idx_v.at[pl.ds(0, nhead)])

    def out_window(j):
        r0 = row_base + CHUNK * lax.rem(j, 2)
        return out_hbm.at[pl.ds(r0, CHUNK), pl.ds(D * lax.div(j, 2), D)]

    def fire_gather(j, s):
        pltpu.async_copy(
            table_hbm.at[idx_v.at[lax.div(j, 2), lax.rem(j, 2)]], bufs.at[s], gsem[s]
        )

    def drain_gather(j, s):
        pltpu.make_async_copy(
            table_hbm.at[idx_v.at[lax.div(j, 2), lax.rem(j, 2)]], bufs.at[s], gsem[s]
        ).wait()

    def fire_wb(j, s):
        pltpu.async_copy(bufs.at[s], out_window(j), wsem[s])

    def drain_wb(j, s):
        pltpu.make_async_copy(bufs.at[s], out_window(j), wsem[s]).wait()

    def step(j, s, drain_prev=True, fire_next=True):
        drain_gather(j, s)
        fire_wb(j, s)
        if drain_prev:
            drain_wb(j - (NBUF - GAHEAD), (s + GAHEAD) % NBUF)
        if fire_next:

            @pl.when(j + GAHEAD < NCHUNK)
            def _():
                fire_gather(j + GAHEAD, (s + GAHEAD) % NBUF)

    for s in range(GAHEAD):
        fire_gather(s, s)
    pltpu.sync_copy(
        ids_hbm.at[pl.ds(nhead, L - nhead), wid], idx_v.at[pl.ds(nhead, L - nhead)]
    )
    step(0, 0, drain_prev=False)
    step(1, 1, drain_prev=False)

    def block(bb, carry):
        j0 = NBUF * bb + 2
        for k in range(NBUF):
            step(j0 + k, (2 + k) % NBUF)
        return carry

    lax.fori_loop(0, (NCHUNK - 2) // NBUF, block, 0)

    drain_wb(NCHUNK - 2, (NCHUNK - 2) % NBUF)
    drain_wb(NCHUNK - 1, (NCHUNK - 1) % NBUF)


def kernel(node_ids, node_features):
    ids = node_ids.T.reshape(L, NW, 2, CHUNK)
    return _gather_kernel(ids, node_features)


# speedup vs baseline: 1.0391x; 1.0391x over previous
import functools

import jax
import jax.numpy as jnp
from jax import lax
from jax.experimental import pallas as pl
from jax.experimental.pallas import tpu as pltpu
from jax.experimental.pallas import tpu_sc as plsc

NC = 2
NS = 16
NW = NC * NS

B, L = 4096, 50
D = 128
TOTAL = B * L
PER_W = TOTAL // NW
CHUNK = 128
NCHUNK = PER_W // CHUNK
NBUF = 6
GAHEAD = 4

_mesh = plsc.VectorSubcoreMesh(
    core_axis_name="c", subcore_axis_name="s", num_cores=NC, num_subcores=NS
)


@functools.partial(
    pl.kernel,
    out_type=jax.ShapeDtypeStruct((B, L * D), jnp.float32),
    mesh=_mesh,
    compiler_params=pltpu.CompilerParams(use_tc_tiling_on_sc=True),
    scratch_types=[
        pltpu.VMEM((NCHUNK, CHUNK), jnp.int32),
        pltpu.VMEM((NBUF, CHUNK, D), jnp.float32),
        tuple(pltpu.SemaphoreType.DMA for _ in range(NBUF)),
        tuple(pltpu.SemaphoreType.DMA for _ in range(NBUF)),
    ],
)
def _gather_kernel(ids_hbm, table_hbm, out_hbm, idx_v, bufs, gsem, wsem):
    wid = lax.axis_index("s") * NC + lax.axis_index("c")
    row_base = wid * (B // NW)
    pltpu.sync_copy(ids_hbm.at[pl.ds(0, GAHEAD), wid], idx_v.at[pl.ds(0, GAHEAD)])

    def out_window(j):
        return out_hbm.at[pl.ds(row_base, CHUNK), pl.ds(D * j, D)]

    def fire_gather(j, s):
        pltpu.async_copy(table_hbm.at[idx_v.at[j]], bufs.at[s], gsem[s])

    def drain_gather(j, s):
        pltpu.make_async_copy(table_hbm.at[idx_v.at[j]], bufs.at[s], gsem[s]).wait()

    def fire_wb(j, s):
        pltpu.async_copy(bufs.at[s], out_window(j), wsem[s])

    def drain_wb(j, s):
        pltpu.make_async_copy(bufs.at[s], out_window(j), wsem[s]).wait()

    def step(j, s, drain_prev=True, fire_next=True):
        drain_gather(j, s)
        fire_wb(j, s)
        if drain_prev:
            drain_wb(j - (NBUF - GAHEAD), (s + GAHEAD) % NBUF)
        if fire_next:

            @pl.when(j + GAHEAD < NCHUNK)
            def _():
                fire_gather(j + GAHEAD, (s + GAHEAD) % NBUF)

    for s in range(GAHEAD):
        fire_gather(s, s)
    pltpu.sync_copy(
        ids_hbm.at[pl.ds(GAHEAD, NCHUNK - GAHEAD), wid],
        idx_v.at[pl.ds(GAHEAD, NCHUNK - GAHEAD)],
    )
    step(0, 0, drain_prev=False)
    step(1, 1, drain_prev=False)

    def block(bb, carry):
        j0 = NBUF * bb + 2
        for k in range(NBUF):
            step(j0 + k, (2 + k) % NBUF)
        return carry

    lax.fori_loop(0, (NCHUNK - 2) // NBUF, block, 0)

    drain_wb(NCHUNK - 2, (NCHUNK - 2) % NBUF)
    drain_wb(NCHUNK - 1, (NCHUNK - 1) % NBUF)


def kernel(node_ids, node_features):
    ids = node_ids.T.reshape(L, NW, CHUNK)
    return _gather_kernel(ids, node_features)
